# baseline (speedup 1.0000x reference)
import functools

import jax
import jax.numpy as jnp
from jax import lax
from jax.experimental import pallas as pl
from jax.experimental.pallas import tpu as pltpu
from jax.experimental.pallas import tpu_sc as plsc

N_NODES = 10000
N_EDGES = 320000
D_FEAT = 128
UNITS = 128

NC = 2
NS = 16
L = 16

D_HALF = UNITS // NC
EG = 100
GROUPS = N_EDGES // EG
GROUPS_PER_TILE = GROUPS // NS
KG = 40
MEGA = GROUPS_PER_TILE // KG
NODE_STEP = 624
NODE_BLK = 640
NSUB = 160
NQ = NODE_BLK // NSUB


def _splat(ref, *idxs):
    return plsc.load_gather(
        ref, [jnp.full((L,), i, jnp.int32) for i in idxs])


def _matmul_body(x_ref, w_ref, o_ref):
    o_ref[...] = jnp.dot(x_ref[...], w_ref[...],
                         preferred_element_type=jnp.float32)


def _tc_matmul(x, w):
    blk = 2000
    return pl.pallas_call(
        _matmul_body,
        grid=(N_NODES // blk,),
        in_specs=[
            pl.BlockSpec((blk, D_FEAT), lambda i: (i, 0)),
            pl.BlockSpec((D_FEAT, UNITS), lambda i: (0, 0)),
        ],
        out_specs=pl.BlockSpec((blk, UNITS), lambda i: (i, 0)),
        out_shape=jax.ShapeDtypeStruct((N_NODES, UNITS), jnp.float32),
    )(x, w)


EDGES_PER_TILE = N_EDGES // NS
EW = 2000
DEGW = EDGES_PER_TILE // EW


def _sc_body(h_hbm, row_hbm, col_hbm, w_hbm, rowf_hbm, wf_hbm,
             bias_hbm, out_hbm,
             g_sh, acc_sh, deg_sh,
             h_v, deg_v, dis_v, row_mk, col_mk, w_mk, rows_v0, rows_v1,
             row_1d, w_1d, bias_v, sem_g0, sem_g1, sem_s0, sem_s1):
    c = lax.axis_index("c")
    s = lax.axis_index("s")
    nbase = s * NODE_STEP
    gbase = s * GROUPS_PER_TILE
    fbase = c * D_HALF

    @pl.loop(0, NODE_BLK // L)
    def _(jj):
        deg_v[pl.ds(jj * L, L)] = jnp.full((L,), 1.0, jnp.float32)

    pltpu.sync_copy(deg_v, deg_sh.at[pl.ds(nbase, NODE_BLK)])
    plsc.subcore_barrier()

    ebase = s * EDGES_PER_TILE

    @pl.loop(0, DEGW)
    def _(mk):
        e0 = ebase + mk * EW
        pltpu.sync_copy(rowf_hbm.at[pl.ds(e0, EW)], row_1d)
        pltpu.sync_copy(wf_hbm.at[pl.ds(e0, EW)], w_1d)
        pltpu.sync_copy(w_1d, deg_sh.at[row_1d], add=True)

    plsc.subcore_barrier()

    pltpu.sync_copy(deg_sh.at[pl.ds(nbase, NODE_BLK)], deg_v)

    @plsc.parallel_loop(0, NODE_BLK // L, unroll=2)
    def _(jj):
        sl = pl.ds(jj * L, L)
        d = deg_v[sl]
        yi = jnp.int32(0x5F3759DF) - (lax.bitcast_convert_type(d, jnp.int32) >> 1)
        y = lax.bitcast_convert_type(yi, jnp.float32)
        for _ in range(3):
            y = y * (1.5 - 0.5 * d * y * y)
        dis_v[sl] = y

    @pl.loop(0, NQ)
    def _(q):
        nb = nbase + q * NSUB
        pltpu.sync_copy(h_hbm.at[pl.ds(nb, NSUB), pl.ds(fbase, D_HALF)],
                        h_v)

        @plsc.parallel_loop(0, NSUB, unroll=2)
        def _(r):
            dsplat = _splat(dis_v, q * NSUB + r)
            for k in range(D_HALF // L):
                sl = pl.ds(k * L, L)
                h_v[r, sl] = h_v[r, sl] * dsplat

        pltpu.sync_copy(h_v, g_sh.at[pl.ds(nb, NSUB)])
        pltpu.sync_copy(h_v, acc_sh.at[pl.ds(nb, NSUB)])

    plsc.subcore_barrier()

    def _scale(buf, j):
        @plsc.parallel_loop(0, EG, unroll=4)
        def _(e):
            ws = _splat(w_mk, j, e)
            for k in range(D_HALF // L):
                sl = pl.ds(k * L, L)
                buf[e, sl] = buf[e, sl] * ws

    def _gath(j, buf, sem):
        pltpu.async_copy(g_sh.at[col_mk.at[j]], buf, sem)

    def _gath_wait(j, buf, sem):
        pltpu.make_async_copy(g_sh.at[col_mk.at[j]], buf, sem).wait()

    def _scat(j, buf, sem):
        pltpu.async_copy(buf, acc_sh.at[row_mk.at[j]], sem, add=True)

    def _scat_wait(j, buf, sem):
        pltpu.make_async_copy(buf, acc_sh.at[row_mk.at[j]], sem).wait()

    @pl.loop(0, MEGA)
    def _(mk):
        g0 = gbase + mk * KG
        pltpu.sync_copy(row_hbm.at[pl.ds(g0, KG)], row_mk)
        pltpu.sync_copy(col_hbm.at[pl.ds(g0, KG)], col_mk)
        pltpu.sync_copy(w_hbm.at[pl.ds(g0, KG)], w_mk)

        _gath(0, rows_v0, sem_g0)
        _gath(1, rows_v1, sem_g1)

        @pl.loop(0, KG - 2, step=2)
        def _(j):
            _gath_wait(j, rows_v0, sem_g0)
            _scale(rows_v0, j)
            _scat(j, rows_v0, sem_s0)
            _gath_wait(j + 1, rows_v1, sem_g1)
            _scale(rows_v1, j + 1)
            _scat(j + 1, rows_v1, sem_s1)
            _scat_wait(j, rows_v0, sem_s0)
            _gath(j + 2, rows_v0, sem_g0)
            _scat_wait(j + 1, rows_v1, sem_s1)
            _gath(j + 3, rows_v1, sem_g1)

        _gath_wait(KG - 2, rows_v0, sem_g0)
        _scale(rows_v0, KG - 2)
        _scat(KG - 2, rows_v0, sem_s0)
        _gath_wait(KG - 1, rows_v1, sem_g1)
        _scale(rows_v1, KG - 1)
        _scat(KG - 1, rows_v1, sem_s1)
        _scat_wait(KG - 2, rows_v0, sem_s0)
        _scat_wait(KG - 1, rows_v1, sem_s1)

    plsc.subcore_barrier()

    pltpu.sync_copy(bias_hbm.at[pl.ds(fbase, D_HALF)], bias_v)
    bvs = [bias_v[pl.ds(k * L, L)] for k in range(D_HALF // L)]

    @pl.loop(0, NQ)
    def _(q):
        nb = nbase + q * NSUB
        pltpu.sync_copy(acc_sh.at[pl.ds(nb, NSUB)], h_v)

        @plsc.parallel_loop(0, NSUB, unroll=2)
        def _(r):
            dsplat = _splat(dis_v, q * NSUB + r)
            for k in range(D_HALF // L):
                sl = pl.ds(k * L, L)
                h_v[r, sl] = h_v[r, sl] * dsplat + bvs[k]

        pltpu.sync_copy(h_v, out_hbm.at[pl.ds(nb, NSUB),
                                        pl.ds(fbase, D_HALF)])


@functools.partial(
    pl.kernel,
    out_type=jax.ShapeDtypeStruct((N_NODES, UNITS), jnp.float32),
    mesh=plsc.VectorSubcoreMesh(core_axis_name="c", subcore_axis_name="s",
                                num_cores=NC, num_subcores=NS),
    compiler_params=pltpu.CompilerParams(needs_layout_passes=False,
                                         use_tc_tiling_on_sc=False),
    scratch_types=[
        pltpu.VMEM_SHARED((N_NODES, D_HALF), jnp.float32),
        pltpu.VMEM_SHARED((N_NODES, D_HALF), jnp.float32),
        pltpu.VMEM_SHARED((N_NODES,), jnp.float32),
        pltpu.VMEM((NSUB, D_HALF), jnp.float32),
        pltpu.VMEM((NODE_BLK,), jnp.float32),
        pltpu.VMEM((NODE_BLK,), jnp.float32),
        pltpu.VMEM((KG, EG), jnp.int32),
        pltpu.VMEM((KG, EG), jnp.int32),
        pltpu.VMEM((KG, EG), jnp.float32),
        pltpu.VMEM((EG, D_HALF), jnp.float32),
        pltpu.VMEM((EG, D_HALF), jnp.float32),
        pltpu.VMEM((EW,), jnp.int32),
        pltpu.VMEM((EW,), jnp.float32),
        pltpu.VMEM((D_HALF,), jnp.float32),
        pltpu.SemaphoreType.DMA,
        pltpu.SemaphoreType.DMA,
        pltpu.SemaphoreType.DMA,
        pltpu.SemaphoreType.DMA,
    ],
)
def _sc_gcn(h_hbm, row_hbm, col_hbm, w_hbm, rowf_hbm, wf_hbm, bias_hbm,
            out_hbm, *scratch):
    _sc_body(h_hbm, row_hbm, col_hbm, w_hbm, rowf_hbm, wf_hbm, bias_hbm,
             out_hbm, *scratch)


def kernel(x, edge_index, edge_weight, kernel, bias):
    rowf = edge_index[0].astype(jnp.int32)
    colf = edge_index[1].astype(jnp.int32)
    wf = edge_weight.astype(jnp.float32)
    row = rowf.reshape(GROUPS, EG)
    col = colf.reshape(GROUPS, EG)
    w = wf.reshape(GROUPS, EG)
    h = _tc_matmul(x, kernel)
    return _sc_gcn(h, row, col, w, rowf, wf, bias)


# speedup vs baseline: 1.0038x; 1.0038x over previous
import functools

import jax
import jax.numpy as jnp
from jax import lax
from jax.experimental import pallas as pl
from jax.experimental.pallas import tpu as pltpu
from jax.experimental.pallas import tpu_sc as plsc

N_NODES = 10000
N_EDGES = 320000
D_FEAT = 128
UNITS = 128

NC = 2
NS = 16
L = 16

D_HALF = UNITS // NC
EG = 100
GROUPS = N_EDGES // EG
GROUPS_PER_TILE = GROUPS // NS
KG = 40
MEGA = GROUPS_PER_TILE // KG
NODE_STEP = 624
NODE_BLK = 640
NSUB = 160
NQ = NODE_BLK // NSUB


def _splat(ref, *idxs):
    return plsc.load_gather(
        ref, [jnp.full((L,), i, jnp.int32) for i in idxs])


def _matmul_body(x_ref, w_ref, o_ref):
    o_ref[...] = jnp.dot(x_ref[...], w_ref[...],
                         preferred_element_type=jnp.float32)


def _tc_matmul(x, w):
    blk = 2000
    return pl.pallas_call(
        _matmul_body,
        grid=(N_NODES // blk,),
        in_specs=[
            pl.BlockSpec((blk, D_FEAT), lambda i: (i, 0)),
            pl.BlockSpec((D_FEAT, UNITS), lambda i: (0, 0)),
        ],
        out_specs=pl.BlockSpec((blk, UNITS), lambda i: (i, 0)),
        out_shape=jax.ShapeDtypeStruct((N_NODES, UNITS), jnp.float32),
    )(x, w)


EDGES_PER_TILE = N_EDGES // NS
EW = 2000
DEGW = EDGES_PER_TILE // EW


def _sc_body(h_hbm, row_hbm, col_hbm, w_hbm, rowf_hbm, wf_hbm,
             bias_hbm, out_hbm,
             g_sh, acc_sh, deg_sh,
             h_v, deg_v, dis_v, row_mk, col_mk, w_mk, rows_v0, rows_v1,
             row_1d, w_1d, bias_v, sem_g0, sem_g1, sem_s0, sem_s1):
    c = lax.axis_index("c")
    s = lax.axis_index("s")
    nbase = s * NODE_STEP
    gbase = s * GROUPS_PER_TILE
    fbase = c * D_HALF

    @pl.loop(0, NODE_BLK // L)
    def _(jj):
        deg_v[pl.ds(jj * L, L)] = jnp.full((L,), 1.0, jnp.float32)

    pltpu.sync_copy(deg_v, deg_sh.at[pl.ds(nbase, NODE_BLK)])
    plsc.subcore_barrier()

    ebase = s * EDGES_PER_TILE

    @pl.loop(0, DEGW)
    def _(mk):
        e0 = ebase + mk * EW
        pltpu.sync_copy(rowf_hbm.at[pl.ds(e0, EW)], row_1d)
        pltpu.sync_copy(wf_hbm.at[pl.ds(e0, EW)], w_1d)
        pltpu.sync_copy(w_1d, deg_sh.at[row_1d], add=True)

    plsc.subcore_barrier()

    pltpu.sync_copy(deg_sh.at[pl.ds(nbase, NODE_BLK)], deg_v)

    @plsc.parallel_loop(0, NODE_BLK // L, unroll=2)
    def _(jj):
        sl = pl.ds(jj * L, L)
        d = deg_v[sl]
        yi = jnp.int32(0x5F3759DF) - (lax.bitcast_convert_type(d, jnp.int32) >> 1)
        y = lax.bitcast_convert_type(yi, jnp.float32)
        for _ in range(3):
            y = y * (1.5 - 0.5 * d * y * y)
        dis_v[sl] = y

    @pl.loop(0, NQ)
    def _(q):
        nb = nbase + q * NSUB
        pltpu.sync_copy(h_hbm.at[pl.ds(nb, NSUB), pl.ds(fbase, D_HALF)],
                        h_v)

        @plsc.parallel_loop(0, NSUB, unroll=2)
        def _(r):
            dsplat = _splat(dis_v, q * NSUB + r)
            for k in range(D_HALF // L):
                sl = pl.ds(k * L, L)
                h_v[r, sl] = h_v[r, sl] * dsplat

        pltpu.sync_copy(h_v, g_sh.at[pl.ds(nb, NSUB)])
        pltpu.sync_copy(h_v, acc_sh.at[pl.ds(nb, NSUB)])

    plsc.subcore_barrier()

    def _scale(buf, j):
        @plsc.parallel_loop(0, EG, unroll=2)
        def _(e):
            ws = _splat(w_mk, j, e)
            for k in range(D_HALF // L):
                sl = pl.ds(k * L, L)
                buf[e, sl] = buf[e, sl] * ws

    def _gath(j, buf, sem):
        pltpu.async_copy(g_sh.at[col_mk.at[j]], buf, sem)

    def _gath_wait(j, buf, sem):
        pltpu.make_async_copy(g_sh.at[col_mk.at[j]], buf, sem).wait()

    def _scat(j, buf, sem):
        pltpu.async_copy(buf, acc_sh.at[row_mk.at[j]], sem, add=True)

    def _scat_wait(j, buf, sem):
        pltpu.make_async_copy(buf, acc_sh.at[row_mk.at[j]], sem).wait()

    @pl.loop(0, MEGA)
    def _(mk):
        g0 = gbase + mk * KG
        pltpu.sync_copy(row_hbm.at[pl.ds(g0, KG)], row_mk)
        pltpu.sync_copy(col_hbm.at[pl.ds(g0, KG)], col_mk)
        pltpu.sync_copy(w_hbm.at[pl.ds(g0, KG)], w_mk)

        _gath(0, rows_v0, sem_g0)
        _gath(1, rows_v1, sem_g1)

        @pl.loop(0, KG - 2, step=2)
        def _(j):
            _gath_wait(j, rows_v0, sem_g0)
            _scale(rows_v0, j)
            _scat(j, rows_v0, sem_s0)
            _gath_wait(j + 1, rows_v1, sem_g1)
            _scale(rows_v1, j + 1)
            _scat(j + 1, rows_v1, sem_s1)
            _scat_wait(j, rows_v0, sem_s0)
            _gath(j + 2, rows_v0, sem_g0)
            _scat_wait(j + 1, rows_v1, sem_s1)
            _gath(j + 3, rows_v1, sem_g1)

        _gath_wait(KG - 2, rows_v0, sem_g0)
        _scale(rows_v0, KG - 2)
        _scat(KG - 2, rows_v0, sem_s0)
        _gath_wait(KG - 1, rows_v1, sem_g1)
        _scale(rows_v1, KG - 1)
        _scat(KG - 1, rows_v1, sem_s1)
        _scat_wait(KG - 2, rows_v0, sem_s0)
        _scat_wait(KG - 1, rows_v1, sem_s1)

    plsc.subcore_barrier()

    pltpu.sync_copy(bias_hbm.at[pl.ds(fbase, D_HALF)], bias_v)
    bvs = [bias_v[pl.ds(k * L, L)] for k in range(D_HALF // L)]

    @pl.loop(0, NQ)
    def _(q):
        nb = nbase + q * NSUB
        pltpu.sync_copy(acc_sh.at[pl.ds(nb, NSUB)], h_v)

        @plsc.parallel_loop(0, NSUB, unroll=2)
        def _(r):
            dsplat = _splat(dis_v, q * NSUB + r)
            for k in range(D_HALF // L):
                sl = pl.ds(k * L, L)
                h_v[r, sl] = h_v[r, sl] * dsplat + bvs[k]

        pltpu.sync_copy(h_v, out_hbm.at[pl.ds(nb, NSUB),
                                        pl.ds(fbase, D_HALF)])


@functools.partial(
    pl.kernel,
    out_type=jax.ShapeDtypeStruct((N_NODES, UNITS), jnp.float32),
    mesh=plsc.VectorSubcoreMesh(core_axis_name="c", subcore_axis_name="s",
                                num_cores=NC, num_subcores=NS),
    compiler_params=pltpu.CompilerParams(needs_layout_passes=False,
                                         use_tc_tiling_on_sc=False),
    scratch_types=[
        pltpu.VMEM_SHARED((N_NODES, D_HALF), jnp.float32),
        pltpu.VMEM_SHARED((N_NODES, D_HALF), jnp.float32),
        pltpu.VMEM_SHARED((N_NODES,), jnp.float32),
        pltpu.VMEM((NSUB, D_HALF), jnp.float32),
        pltpu.VMEM((NODE_BLK,), jnp.float32),
        pltpu.VMEM((NODE_BLK,), jnp.float32),
        pltpu.VMEM((KG, EG), jnp.int32),
        pltpu.VMEM((KG, EG), jnp.int32),
        pltpu.VMEM((KG, EG), jnp.float32),
        pltpu.VMEM((EG, D_HALF), jnp.float32),
        pltpu.VMEM((EG, D_HALF), jnp.float32),
        pltpu.VMEM((EW,), jnp.int32),
        pltpu.VMEM((EW,), jnp.float32),
        pltpu.VMEM((D_HALF,), jnp.float32),
        pltpu.SemaphoreType.DMA,
        pltpu.SemaphoreType.DMA,
        pltpu.SemaphoreType.DMA,
        pltpu.SemaphoreType.DMA,
    ],
)
def _sc_gcn(h_hbm, row_hbm, col_hbm, w_hbm, rowf_hbm, wf_hbm, bias_hbm,
            out_hbm, *scratch):
    _sc_body(h_hbm, row_hbm, col_hbm, w_hbm, rowf_hbm, wf_hbm, bias_hbm,
             out_hbm, *scratch)


def kernel(x, edge_index, edge_weight, kernel, bias):
    rowf = edge_index[0].astype(jnp.int32)
    colf = edge_index[1].astype(jnp.int32)
    wf = edge_weight.astype(jnp.float32)
    row = rowf.reshape(GROUPS, EG)
    col = colf.reshape(GROUPS, EG)
    w = wf.reshape(GROUPS, EG)
    h = _tc_matmul(x, kernel)
    return _sc_gcn(h, row, col, w, rowf, wf, bias)


# speedup vs baseline: 1.1274x; 1.1232x over previous
import functools

import jax
import jax.numpy as jnp
from jax import lax
from jax.experimental import pallas as pl
from jax.experimental.pallas import tpu as pltpu
from jax.experimental.pallas import tpu_sc as plsc

N_NODES = 10000
N_EDGES = 320000
D_FEAT = 128
UNITS = 128

NC = 2
NS = 16
L = 16

D_HALF = UNITS // NC
EG = 100
GROUPS = N_EDGES // EG
GROUPS_PER_TILE = GROUPS // NS
KG = 40
MEGA = GROUPS_PER_TILE // KG
NODE_STEP = 624
NODE_BLK = 640
NSUB = 160
NQ = NODE_BLK // NSUB


def _splat(ref, *idxs):
    return plsc.load_gather(
        ref, [jnp.full((L,), i, jnp.int32) for i in idxs])


def _matmul_body(x_ref, w_ref, o_ref):
    o_ref[...] = jnp.dot(x_ref[...], w_ref[...],
                         preferred_element_type=jnp.float32)


def _tc_matmul(x, w):
    blk = 2000
    return pl.pallas_call(
        _matmul_body,
        grid=(N_NODES // blk,),
        in_specs=[
            pl.BlockSpec((blk, D_FEAT), lambda i: (i, 0)),
            pl.BlockSpec((D_FEAT, UNITS), lambda i: (0, 0)),
        ],
        out_specs=pl.BlockSpec((blk, UNITS), lambda i: (i, 0)),
        out_shape=jax.ShapeDtypeStruct((N_NODES, UNITS), jnp.float32),
    )(x, w)


EDGES_PER_TILE = N_EDGES // NS
EW = 4000
DEGW = EDGES_PER_TILE // EW


def _sc_body(h_hbm, row_hbm, col_hbm, w_hbm, rowf_hbm, wf_hbm,
             bias_hbm, out_hbm,
             g_sh, acc_sh, deg_sh,
             h_v, deg_v, dis_v, row_mk, col_mk, w_mk, rows_v0, rows_v1,
             row_1d, w_1d, bias_v, sem_g0, sem_g1, sem_s0, sem_s1):
    c = lax.axis_index("c")
    s = lax.axis_index("s")
    nbase = s * NODE_STEP
    gbase = s * GROUPS_PER_TILE
    fbase = c * D_HALF

    @pl.loop(0, NODE_BLK // L)
    def _(jj):
        deg_v[pl.ds(jj * L, L)] = jnp.full((L,), 1.0, jnp.float32)

    pltpu.sync_copy(deg_v, deg_sh.at[pl.ds(nbase, NODE_BLK)])
    plsc.subcore_barrier()

    ebase = s * EDGES_PER_TILE

    @pl.loop(0, DEGW)
    def _(mk):
        e0 = ebase + mk * EW
        pltpu.sync_copy(rowf_hbm.at[pl.ds(e0, EW)], row_1d)
        pltpu.sync_copy(wf_hbm.at[pl.ds(e0, EW)], w_1d)
        pltpu.sync_copy(w_1d, deg_sh.at[row_1d], add=True)

    plsc.subcore_barrier()

    pltpu.sync_copy(deg_sh.at[pl.ds(nbase, NODE_BLK)], deg_v)

    @plsc.parallel_loop(0, NODE_BLK // L, unroll=2)
    def _(jj):
        sl = pl.ds(jj * L, L)
        d = deg_v[sl]
        yi = jnp.int32(0x5F3759DF) - (lax.bitcast_convert_type(d, jnp.int32) >> 1)
        y = lax.bitcast_convert_type(yi, jnp.float32)
        for _ in range(3):
            y = y * (1.5 - 0.5 * d * y * y)
        dis_v[sl] = y

    @pl.loop(0, NQ)
    def _(q):
        nb = nbase + q * NSUB
        pltpu.sync_copy(h_hbm.at[pl.ds(nb, NSUB), pl.ds(fbase, D_HALF)],
                        h_v)

        @plsc.parallel_loop(0, NSUB, unroll=2)
        def _(r):
            dsplat = _splat(dis_v, q * NSUB + r)
            for k in range(D_HALF // L):
                sl = pl.ds(k * L, L)
                h_v[r, sl] = h_v[r, sl] * dsplat

        pltpu.sync_copy(h_v, g_sh.at[pl.ds(nb, NSUB)])
        pltpu.sync_copy(h_v, acc_sh.at[pl.ds(nb, NSUB)])

    plsc.subcore_barrier()

    def _scale(buf, j):
        @plsc.parallel_loop(0, EG, unroll=2)
        def _(e):
            ws = _splat(w_mk, j, e)
            for k in range(D_HALF // L):
                sl = pl.ds(k * L, L)
                buf[e, sl] = buf[e, sl] * ws

    def _gath(j, buf, sem):
        pltpu.async_copy(g_sh.at[col_mk.at[j]], buf, sem)

    def _gath_wait(j, buf, sem):
        pltpu.make_async_copy(g_sh.at[col_mk.at[j]], buf, sem).wait()

    def _scat(j, buf, sem):
        pltpu.async_copy(buf, acc_sh.at[row_mk.at[j]], sem, add=True)

    def _scat_wait(j, buf, sem):
        pltpu.make_async_copy(buf, acc_sh.at[row_mk.at[j]], sem).wait()

    @pl.loop(0, MEGA)
    def _(mk):
        g0 = gbase + mk * KG
        pltpu.sync_copy(row_hbm.at[pl.ds(g0, KG)], row_mk)
        pltpu.sync_copy(col_hbm.at[pl.ds(g0, KG)], col_mk)
        pltpu.sync_copy(w_hbm.at[pl.ds(g0, KG)], w_mk)

        _gath(0, rows_v0, sem_g0)

        @pl.loop(0, KG, step=2)
        def _(j):
            _gath_wait(j, rows_v0, sem_g0)
            _gath(j + 1, rows_v1, sem_g1)
            _scale(rows_v0, j)
            pltpu.sync_copy(rows_v0, acc_sh.at[row_mk.at[j]], add=True)

            jn = jnp.minimum(j + 2, KG - 1)
            _gath_wait(j + 1, rows_v1, sem_g1)
            _gath(jn, rows_v0, sem_g0)
            _scale(rows_v1, j + 1)
            pltpu.sync_copy(rows_v1, acc_sh.at[row_mk.at[j + 1]], add=True)

        _gath_wait(KG - 1, rows_v0, sem_g0)

    plsc.subcore_barrier()

    pltpu.sync_copy(bias_hbm.at[pl.ds(fbase, D_HALF)], bias_v)
    bvs = [bias_v[pl.ds(k * L, L)] for k in range(D_HALF // L)]

    @pl.loop(0, NQ)
    def _(q):
        nb = nbase + q * NSUB
        pltpu.sync_copy(acc_sh.at[pl.ds(nb, NSUB)], h_v)

        @plsc.parallel_loop(0, NSUB, unroll=2)
        def _(r):
            dsplat = _splat(dis_v, q * NSUB + r)
            for k in range(D_HALF // L):
                sl = pl.ds(k * L, L)
                h_v[r, sl] = h_v[r, sl] * dsplat + bvs[k]

        pltpu.sync_copy(h_v, out_hbm.at[pl.ds(nb, NSUB),
                                        pl.ds(fbase, D_HALF)])


@functools.partial(
    pl.kernel,
    out_type=jax.ShapeDtypeStruct((N_NODES, UNITS), jnp.float32),
    mesh=plsc.VectorSubcoreMesh(core_axis_name="c", subcore_axis_name="s",
                                num_cores=NC, num_subcores=NS),
    compiler_params=pltpu.CompilerParams(needs_layout_passes=False,
                                         use_tc_tiling_on_sc=False),
    scratch_types=[
        pltpu.VMEM_SHARED((N_NODES, D_HALF), jnp.float32),
        pltpu.VMEM_SHARED((N_NODES, D_HALF), jnp.float32),
        pltpu.VMEM_SHARED((N_NODES,), jnp.float32),
        pltpu.VMEM((NSUB, D_HALF), jnp.float32),
        pltpu.VMEM((NODE_BLK,), jnp.float32),
        pltpu.VMEM((NODE_BLK,), jnp.float32),
        pltpu.VMEM((KG, EG), jnp.int32),
        pltpu.VMEM((KG, EG), jnp.int32),
        pltpu.VMEM((KG, EG), jnp.float32),
        pltpu.VMEM((EG, D_HALF), jnp.float32),
        pltpu.VMEM((EG, D_HALF), jnp.float32),
        pltpu.VMEM((EW,), jnp.int32),
        pltpu.VMEM((EW,), jnp.float32),
        pltpu.VMEM((D_HALF,), jnp.float32),
        pltpu.SemaphoreType.DMA,
        pltpu.SemaphoreType.DMA,
        pltpu.SemaphoreType.DMA,
        pltpu.SemaphoreType.DMA,
    ],
)
def _sc_gcn(h_hbm, row_hbm, col_hbm, w_hbm, rowf_hbm, wf_hbm, bias_hbm,
            out_hbm, *scratch):
    _sc_body(h_hbm, row_hbm, col_hbm, w_hbm, rowf_hbm, wf_hbm, bias_hbm,
             out_hbm, *scratch)


def kernel(x, edge_index, edge_weight, kernel, bias):
    rowf = edge_index[0].astype(jnp.int32)
    colf = edge_index[1].astype(jnp.int32)
    wf = edge_weight.astype(jnp.float32)
    row = rowf.reshape(GROUPS, EG)
    col = colf.reshape(GROUPS, EG)
    w = wf.reshape(GROUPS, EG)
    h = _tc_matmul(x, kernel)
    return _sc_gcn(h, row, col, w, rowf, wf, bias)


# speedup vs baseline: 1.1293x; 1.0017x over previous
import functools

import jax
import jax.numpy as jnp
from jax import lax
from jax.experimental import pallas as pl
from jax.experimental.pallas import tpu as pltpu
from jax.experimental.pallas import tpu_sc as plsc

N_NODES = 10000
N_EDGES = 320000
D_FEAT = 128
UNITS = 128

NC = 2
NS = 16
L = 16

D_HALF = UNITS // NC
EG = 100
GROUPS = N_EDGES // EG
GROUPS_PER_TILE = GROUPS // NS
KG = 40
MEGA = GROUPS_PER_TILE // KG
NODE_STEP = 624
NODE_BLK = 640
NSUB = 160
NQ = NODE_BLK // NSUB


def _splat(ref, *idxs):
    return plsc.load_gather(
        ref, [jnp.full((L,), i, jnp.int32) for i in idxs])


def _matmul_body(x_ref, w_ref, o_ref):
    o_ref[...] = jnp.dot(x_ref[...], w_ref[...],
                         preferred_element_type=jnp.float32)


def _tc_matmul(x, w):
    blk = 2000
    return pl.pallas_call(
        _matmul_body,
        grid=(N_NODES // blk,),
        in_specs=[
            pl.BlockSpec((blk, D_FEAT), lambda i: (i, 0)),
            pl.BlockSpec((D_FEAT, UNITS), lambda i: (0, 0)),
        ],
        out_specs=pl.BlockSpec((blk, UNITS), lambda i: (i, 0)),
        out_shape=jax.ShapeDtypeStruct((N_NODES, UNITS), jnp.float32),
    )(x, w)


EDGES_PER_TILE = N_EDGES // NS
EW = 4000
DEGW = EDGES_PER_TILE // EW


def _sc_body(h_hbm, row_hbm, col_hbm, w_hbm, rowf_hbm, wf_hbm,
             bias_hbm, out_hbm,
             g_sh, acc_sh, deg_sh,
             h_v, deg_v, dis_v, row_mk, col_mk, w_mk, rows_v0, rows_v1,
             row_1d, w_1d, bias_v, sem_g0, sem_g1, sem_s0, sem_s1):
    c = lax.axis_index("c")
    s = lax.axis_index("s")
    nbase = s * NODE_STEP
    gbase = s * GROUPS_PER_TILE
    fbase = c * D_HALF

    _ns = jax.named_scope
    @pl.loop(0, NODE_BLK // L)
    def _(jj):
        deg_v[pl.ds(jj * L, L)] = jnp.full((L,), 1.0, jnp.float32)

    pltpu.sync_copy(deg_v, deg_sh.at[pl.ds(nbase, NODE_BLK)])
    plsc.subcore_barrier()

    ebase = s * EDGES_PER_TILE

    _p1 = _ns("phase1_deg"); _p1.__enter__()

    @pl.loop(0, DEGW)
    def _(mk):
        e0 = ebase + mk * EW
        pltpu.sync_copy(rowf_hbm.at[pl.ds(e0, EW)], row_1d)
        pltpu.sync_copy(wf_hbm.at[pl.ds(e0, EW)], w_1d)
        pltpu.sync_copy(w_1d, deg_sh.at[row_1d], add=True)

    _p1.__exit__(None, None, None)
    plsc.subcore_barrier()

    _p2 = _ns("phase2_rsqrt"); _p2.__enter__()
    pltpu.sync_copy(deg_sh.at[pl.ds(nbase, NODE_BLK)], deg_v)

    @plsc.parallel_loop(0, NODE_BLK // L, unroll=2)
    def _(jj):
        sl = pl.ds(jj * L, L)
        d = deg_v[sl]
        yi = jnp.int32(0x5F3759DF) - (lax.bitcast_convert_type(d, jnp.int32) >> 1)
        y = lax.bitcast_convert_type(yi, jnp.float32)
        for _ in range(3):
            y = y * (1.5 - 0.5 * d * y * y)
        dis_v[sl] = y

    _p2.__exit__(None, None, None)
    _p3 = _ns("phase3_stage"); _p3.__enter__()
    @pl.loop(0, NQ)
    def _(q):
        nb = nbase + q * NSUB
        pltpu.sync_copy(h_hbm.at[pl.ds(nb, NSUB), pl.ds(fbase, D_HALF)],
                        h_v)

        @plsc.parallel_loop(0, NSUB, unroll=2)
        def _(r):
            dsplat = _splat(dis_v, q * NSUB + r)
            for k in range(D_HALF // L):
                sl = pl.ds(k * L, L)
                h_v[r, sl] = h_v[r, sl] * dsplat

        pltpu.sync_copy(h_v, g_sh.at[pl.ds(nb, NSUB)])
        pltpu.sync_copy(h_v, acc_sh.at[pl.ds(nb, NSUB)])

    _p3.__exit__(None, None, None)
    plsc.subcore_barrier()

    _p4 = _ns("phase4_edges"); _p4.__enter__()
    def _scale(buf, j):
        @plsc.parallel_loop(0, EG, unroll=2)
        def _(e):
            ws = _splat(w_mk, j, e)
            for k in range(D_HALF // L):
                sl = pl.ds(k * L, L)
                buf[e, sl] = buf[e, sl] * ws

    def _gath(j, buf, sem):
        pltpu.async_copy(g_sh.at[col_mk.at[j]], buf, sem)

    def _gath_wait(j, buf, sem):
        pltpu.make_async_copy(g_sh.at[col_mk.at[j]], buf, sem).wait()

    def _scat(j, buf, sem):
        pltpu.async_copy(buf, acc_sh.at[row_mk.at[j]], sem, add=True)

    def _scat_wait(j, buf, sem):
        pltpu.make_async_copy(buf, acc_sh.at[row_mk.at[j]], sem).wait()

    @pl.loop(0, MEGA)
    def _(mk):
        g0 = gbase + mk * KG
        pltpu.sync_copy(row_hbm.at[pl.ds(g0, KG)], row_mk)
        pltpu.sync_copy(col_hbm.at[pl.ds(g0, KG)], col_mk)
        pltpu.sync_copy(w_hbm.at[pl.ds(g0, KG)], w_mk)

        _gath(0, rows_v0, sem_g0)

        @pl.loop(0, KG, step=2)
        def _(j):
            _gath_wait(j, rows_v0, sem_g0)
            _gath(j + 1, rows_v1, sem_g1)
            _scale(rows_v0, j)
            pltpu.sync_copy(rows_v0, acc_sh.at[row_mk.at[j]], add=True)

            jn = jnp.minimum(j + 2, KG - 1)
            _gath_wait(j + 1, rows_v1, sem_g1)
            _gath(jn, rows_v0, sem_g0)
            _scale(rows_v1, j + 1)
            pltpu.sync_copy(rows_v1, acc_sh.at[row_mk.at[j + 1]], add=True)

        _gath_wait(KG - 1, rows_v0, sem_g0)

    _p4.__exit__(None, None, None)
    plsc.subcore_barrier()

    _p5 = _ns("phase5_writeback"); _p5.__enter__()
    pltpu.sync_copy(bias_hbm.at[pl.ds(fbase, D_HALF)], bias_v)
    bvs = [bias_v[pl.ds(k * L, L)] for k in range(D_HALF // L)]

    @pl.loop(0, NQ)
    def _(q):
        nb = nbase + q * NSUB
        pltpu.sync_copy(acc_sh.at[pl.ds(nb, NSUB)], h_v)

        @plsc.parallel_loop(0, NSUB, unroll=2)
        def _(r):
            dsplat = _splat(dis_v, q * NSUB + r)
            for k in range(D_HALF // L):
                sl = pl.ds(k * L, L)
                h_v[r, sl] = h_v[r, sl] * dsplat + bvs[k]

        pltpu.sync_copy(h_v, out_hbm.at[pl.ds(nb, NSUB),
                                        pl.ds(fbase, D_HALF)])

    _p5.__exit__(None, None, None)


@functools.partial(
    pl.kernel,
    out_type=jax.ShapeDtypeStruct((N_NODES, UNITS), jnp.float32),
    mesh=plsc.VectorSubcoreMesh(core_axis_name="c", subcore_axis_name="s",
                                num_cores=NC, num_subcores=NS),
    compiler_params=pltpu.CompilerParams(needs_layout_passes=False,
                                         use_tc_tiling_on_sc=False),
    scratch_types=[
        pltpu.VMEM_SHARED((N_NODES, D_HALF), jnp.float32),
        pltpu.VMEM_SHARED((N_NODES, D_HALF), jnp.float32),
        pltpu.VMEM_SHARED((N_NODES,), jnp.float32),
        pltpu.VMEM((NSUB, D_HALF), jnp.float32),
        pltpu.VMEM((NODE_BLK,), jnp.float32),
        pltpu.VMEM((NODE_BLK,), jnp.float32),
        pltpu.VMEM((KG, EG), jnp.int32),
        pltpu.VMEM((KG, EG), jnp.int32),
        pltpu.VMEM((KG, EG), jnp.float32),
        pltpu.VMEM((EG, D_HALF), jnp.float32),
        pltpu.VMEM((EG, D_HALF), jnp.float32),
        pltpu.VMEM((EW,), jnp.int32),
        pltpu.VMEM((EW,), jnp.float32),
        pltpu.VMEM((D_HALF,), jnp.float32),
        pltpu.SemaphoreType.DMA,
        pltpu.SemaphoreType.DMA,
        pltpu.SemaphoreType.DMA,
        pltpu.SemaphoreType.DMA,
    ],
)
def _sc_gcn(h_hbm, row_hbm, col_hbm, w_hbm, rowf_hbm, wf_hbm, bias_hbm,
            out_hbm, *scratch):
    _sc_body(h_hbm, row_hbm, col_hbm, w_hbm, rowf_hbm, wf_hbm, bias_hbm,
             out_hbm, *scratch)


def kernel(x, edge_index, edge_weight, kernel, bias):
    rowf = edge_index[0].astype(jnp.int32)
    colf = edge_index[1].astype(jnp.int32)
    wf = edge_weight.astype(jnp.float32)
    row = rowf.reshape(GROUPS, EG)
    col = colf.reshape(GROUPS, EG)
    w = wf.reshape(GROUPS, EG)
    h = _tc_matmul(x, kernel)
    return _sc_gcn(h, row, col, w, rowf, wf, bias)
